# baseline (device time: 65443 ns/iter reference)
import jax
import jax.numpy as jnp
from jax import lax
from jax.experimental import pallas as pl
from jax.experimental.pallas import tpu as pltpu

N_DEV = 16
M_PER = 128
SUBS = 8


def kernel(x, w_mat):
    k_total, m_loc = x.shape
    k_loc, n = w_mat.shape
    nh = n // 2
    ns = nh // SUBS

    def body(x_ref, w_ref, out_ref, commR, commL, sendR, recvR, sendL, recvL):
        my = lax.axis_index("i")
        left = lax.rem(my + N_DEV - 1, N_DEV)
        right = lax.rem(my + 1, N_DEV)

        barrier_sem = pltpu.get_barrier_semaphore()
        for nbr in (left, right):
            pl.semaphore_signal(
                barrier_sem, inc=1,
                device_id=(nbr,), device_id_type=pl.DeviceIdType.MESH,
            )
        pl.semaphore_wait(barrier_sem, 2)

        def partial(c, lo, dtype=jnp.bfloat16):
            xs = x_ref[pl.ds(c * M_PER, M_PER), :]
            r = lax.dot_general(
                xs, w_ref[:, lo:lo + nh], (((1,), (0,)), ((), ())),
                preferred_element_type=jnp.float32,
            )
            return r.astype(dtype)

        def mkR(s, j):
            return pltpu.make_async_remote_copy(
                src_ref=commR.at[s, :, pl.ds(j * ns, ns)],
                dst_ref=commR.at[s + 1, :, pl.ds(j * ns, ns)],
                send_sem=sendR.at[s, j],
                recv_sem=recvR.at[s, j],
                device_id=(right,),
                device_id_type=pl.DeviceIdType.MESH,
            )

        def mkL(s, j):
            return pltpu.make_async_remote_copy(
                src_ref=commL.at[s, :, pl.ds(j * ns, ns)],
                dst_ref=commL.at[s + 1, :, pl.ds(j * ns, ns)],
                send_sem=sendL.at[s, j],
                recv_sem=recvL.at[s, j],
                device_id=(left,),
                device_id_type=pl.DeviceIdType.MESH,
            )

        commR[0, :, :] = partial(lax.rem(my + N_DEV - 1, N_DEV), 0)
        commL[0, :, :] = partial(lax.rem(my + 1, N_DEV), nh)
        for j in range(SUBS):
            mkR(0, j).start()
            mkL(0, j).start()

        for s in range(N_DEV - 1):
            last = s == N_DEV - 2
            dt = jnp.float32 if last else jnp.bfloat16
            pR = partial(lax.rem(my + 2 * N_DEV - 2 - s, N_DEV), 0, dt)
            pL = partial(lax.rem(my + 2 + s, N_DEV), nh, dt)
            for j in range(SUBS):
                sl = slice(j * ns, (j + 1) * ns)
                mkR(s, j).wait_recv()
                if not last:
                    commR[s + 1, :, sl] = pR[:, sl] + commR[s + 1, :, sl]
                    mkR(s + 1, j).start()
                else:
                    out_ref[:, sl] = pR[:, sl] + commR[s + 1, :, sl].astype(jnp.float32)
                mkL(s, j).wait_recv()
                if not last:
                    commL[s + 1, :, sl] = pL[:, sl] + commL[s + 1, :, sl]
                    mkL(s + 1, j).start()
                else:
                    out_ref[:, nh + j * ns:nh + (j + 1) * ns] = (
                        pL[:, sl] + commL[s + 1, :, sl].astype(jnp.float32)
                    )

        for s in range(N_DEV - 1):
            for j in range(SUBS):
                mkR(s, j).wait_send()
                mkL(s, j).wait_send()

    return pl.pallas_call(
        body,
        out_shape=jax.ShapeDtypeStruct((M_PER, n), jnp.float32),
        in_specs=[
            pl.BlockSpec(memory_space=pltpu.VMEM),
            pl.BlockSpec(memory_space=pltpu.VMEM),
        ],
        out_specs=pl.BlockSpec(memory_space=pltpu.VMEM),
        scratch_shapes=[
            pltpu.VMEM((N_DEV, M_PER, nh), jnp.bfloat16),
            pltpu.VMEM((N_DEV, M_PER, nh), jnp.bfloat16),
            pltpu.SemaphoreType.DMA((N_DEV - 1, SUBS)),
            pltpu.SemaphoreType.DMA((N_DEV - 1, SUBS)),
            pltpu.SemaphoreType.DMA((N_DEV - 1, SUBS)),
            pltpu.SemaphoreType.DMA((N_DEV - 1, SUBS)),
        ],
        compiler_params=pltpu.CompilerParams(collective_id=0),
    )(x, w_mat)


# device time: 65122 ns/iter; 1.0049x vs baseline; 1.0049x over previous
import jax
import jax.numpy as jnp
from jax import lax
from jax.experimental import pallas as pl
from jax.experimental.pallas import tpu as pltpu

N_DEV = 16
M_PER = 128
SUBS = 4


def kernel(x, w_mat):
    k_total, m_loc = x.shape
    k_loc, n = w_mat.shape
    nh = n // 2
    mr = M_PER // SUBS

    def body(x_ref, w_ref, out_ref, commR, commL, sendR, recvR, sendL, recvL):
        my = lax.axis_index("i")
        left = lax.rem(my + N_DEV - 1, N_DEV)
        right = lax.rem(my + 1, N_DEV)

        barrier_sem = pltpu.get_barrier_semaphore()
        for nbr in (left, right):
            pl.semaphore_signal(
                barrier_sem, inc=1,
                device_id=(nbr,), device_id_type=pl.DeviceIdType.MESH,
            )
        pl.semaphore_wait(barrier_sem, 2)

        def partial(c, lo, dtype=jnp.bfloat16):
            xs = x_ref[pl.ds(c * M_PER, M_PER), :]
            r = lax.dot_general(
                xs, w_ref[:, lo:lo + nh], (((1,), (0,)), ((), ())),
                preferred_element_type=jnp.float32,
            )
            return r.astype(dtype)

        def mkR(s, j):
            return pltpu.make_async_remote_copy(
                src_ref=commR.at[s, pl.ds(j * mr, mr), :],
                dst_ref=commR.at[s + 1, pl.ds(j * mr, mr), :],
                send_sem=sendR.at[s, j],
                recv_sem=recvR.at[s, j],
                device_id=(right,),
                device_id_type=pl.DeviceIdType.MESH,
            )

        def mkL(s, j):
            return pltpu.make_async_remote_copy(
                src_ref=commL.at[s, pl.ds(j * mr, mr), :],
                dst_ref=commL.at[s + 1, pl.ds(j * mr, mr), :],
                send_sem=sendL.at[s, j],
                recv_sem=recvL.at[s, j],
                device_id=(left,),
                device_id_type=pl.DeviceIdType.MESH,
            )

        commR[0, :, :] = partial(lax.rem(my + N_DEV - 1, N_DEV), 0)
        commL[0, :, :] = partial(lax.rem(my + 1, N_DEV), nh)
        for j in range(SUBS):
            mkR(0, j).start()
            mkL(0, j).start()

        for s in range(N_DEV - 1):
            last = s == N_DEV - 2
            dt = jnp.float32 if last else jnp.bfloat16
            pR = partial(lax.rem(my + 2 * N_DEV - 2 - s, N_DEV), 0, dt)
            pL = partial(lax.rem(my + 2 + s, N_DEV), nh, dt)
            for j in range(SUBS):
                rw = slice(j * mr, (j + 1) * mr)
                mkR(s, j).wait_recv()
                if not last:
                    commR[s + 1, rw, :] = pR[rw, :] + commR[s + 1, rw, :]
                    mkR(s + 1, j).start()
                else:
                    out_ref[rw, 0:nh] = pR[rw, :] + commR[s + 1, rw, :].astype(jnp.float32)
                mkL(s, j).wait_recv()
                if not last:
                    commL[s + 1, rw, :] = pL[rw, :] + commL[s + 1, rw, :]
                    mkL(s + 1, j).start()
                else:
                    out_ref[rw, nh:n] = (
                        pL[rw, :] + commL[s + 1, rw, :].astype(jnp.float32)
                    )

        for s in range(N_DEV - 1):
            for j in range(SUBS):
                mkR(s, j).wait_send()
                mkL(s, j).wait_send()

    return pl.pallas_call(
        body,
        out_shape=jax.ShapeDtypeStruct((M_PER, n), jnp.float32),
        in_specs=[
            pl.BlockSpec(memory_space=pltpu.VMEM),
            pl.BlockSpec(memory_space=pltpu.VMEM),
        ],
        out_specs=pl.BlockSpec(memory_space=pltpu.VMEM),
        scratch_shapes=[
            pltpu.VMEM((N_DEV, M_PER, nh), jnp.bfloat16),
            pltpu.VMEM((N_DEV, M_PER, nh), jnp.bfloat16),
            pltpu.SemaphoreType.DMA((N_DEV - 1, SUBS)),
            pltpu.SemaphoreType.DMA((N_DEV - 1, SUBS)),
            pltpu.SemaphoreType.DMA((N_DEV - 1, SUBS)),
            pltpu.SemaphoreType.DMA((N_DEV - 1, SUBS)),
        ],
        compiler_params=pltpu.CompilerParams(collective_id=0),
    )(x, w_mat)


# device time: 52236 ns/iter; 1.2528x vs baseline; 1.2467x over previous
import jax
import jax.numpy as jnp
from jax import lax
from jax.experimental import pallas as pl
from jax.experimental.pallas import tpu as pltpu

N_DEV = 16
M_PER = 128
SUBS = 4


def kernel(x, w_mat):
    k_total, m_loc = x.shape
    k_loc, n = w_mat.shape
    nh = n // 2
    mr = M_PER // SUBS

    def body(x_ref, w_ref, out_ref, commR, commL, sendR, recvR, sendL, recvL):
        my = lax.axis_index("i")

        def ring2dev(rr):
            rr = lax.rem(rr + 64, N_DEV)
            q = lax.div(rr, 4)
            m = lax.rem(rr, 4)
            z = jnp.where(lax.rem(q, 2) == 0, m, 3 - m)
            return 4 * z + q

        p = lax.rem(my, 4)
        z = lax.div(my, 4)
        r = 4 * p + jnp.where(lax.rem(p, 2) == 0, z, 3 - z)
        left = ring2dev(r - 1)
        right = ring2dev(r + 1)

        barrier_sem = pltpu.get_barrier_semaphore()
        for nbr in (left, right):
            pl.semaphore_signal(
                barrier_sem, inc=1,
                device_id=(nbr,), device_id_type=pl.DeviceIdType.MESH,
            )
        pl.semaphore_wait(barrier_sem, 2)

        def partial(c, lo, dtype=jnp.bfloat16):
            xs = x_ref[pl.ds(c * M_PER, M_PER), :]
            res = lax.dot_general(
                xs, w_ref[:, lo:lo + nh], (((1,), (0,)), ((), ())),
                preferred_element_type=jnp.float32,
            )
            return res.astype(dtype)

        def mkR(s, j):
            return pltpu.make_async_remote_copy(
                src_ref=commR.at[s, pl.ds(j * mr, mr), :],
                dst_ref=commR.at[s + 1, pl.ds(j * mr, mr), :],
                send_sem=sendR.at[s, j],
                recv_sem=recvR.at[s, j],
                device_id=(right,),
                device_id_type=pl.DeviceIdType.MESH,
            )

        def mkL(s, j):
            return pltpu.make_async_remote_copy(
                src_ref=commL.at[s, pl.ds(j * mr, mr), :],
                dst_ref=commL.at[s + 1, pl.ds(j * mr, mr), :],
                send_sem=sendL.at[s, j],
                recv_sem=recvL.at[s, j],
                device_id=(left,),
                device_id_type=pl.DeviceIdType.MESH,
            )

        def cR(k):
            return ring2dev(r - 1 - k)

        def cL(k):
            return ring2dev(r + 1 + k)

        commR[0, :, :] = partial(cR(0), 0)
        commL[0, :, :] = partial(cL(0), nh)
        for j in range(SUBS):
            mkR(0, j).start()
            mkL(0, j).start()
        pR = partial(cR(1), 0)
        pL = partial(cL(1), nh)

        for s in range(N_DEV - 1):
            last = s == N_DEV - 2
            pR_nxt = pL_nxt = None
            for j in range(SUBS):
                rw = slice(j * mr, (j + 1) * mr)
                mkR(s, j).wait_recv()
                if not last:
                    commR[s + 1, rw, :] = pR[rw, :] + commR[s + 1, rw, :]
                    mkR(s + 1, j).start()
                else:
                    out_ref[rw, 0:nh] = (
                        pR[rw, :].astype(jnp.float32)
                        + commR[s + 1, rw, :].astype(jnp.float32)
                    )
                mkL(s, j).wait_recv()
                if not last:
                    commL[s + 1, rw, :] = pL[rw, :] + commL[s + 1, rw, :]
                    mkL(s + 1, j).start()
                else:
                    out_ref[rw, nh:n] = (
                        pL[rw, :].astype(jnp.float32)
                        + commL[s + 1, rw, :].astype(jnp.float32)
                    )
                if j == 0 and not last:
                    pR_nxt = partial(cR(s + 2), 0)
                    pL_nxt = partial(cL(s + 2), nh)
            if not last:
                pR, pL = pR_nxt, pL_nxt

        for s in range(N_DEV - 1):
            for j in range(SUBS):
                mkR(s, j).wait_send()
                mkL(s, j).wait_send()

    return pl.pallas_call(
        body,
        out_shape=jax.ShapeDtypeStruct((M_PER, n), jnp.float32),
        in_specs=[
            pl.BlockSpec(memory_space=pltpu.VMEM),
            pl.BlockSpec(memory_space=pltpu.VMEM),
        ],
        out_specs=pl.BlockSpec(memory_space=pltpu.VMEM),
        scratch_shapes=[
            pltpu.VMEM((N_DEV, M_PER, nh), jnp.bfloat16),
            pltpu.VMEM((N_DEV, M_PER, nh), jnp.bfloat16),
            pltpu.SemaphoreType.DMA((N_DEV - 1, SUBS)),
            pltpu.SemaphoreType.DMA((N_DEV - 1, SUBS)),
            pltpu.SemaphoreType.DMA((N_DEV - 1, SUBS)),
            pltpu.SemaphoreType.DMA((N_DEV - 1, SUBS)),
        ],
        compiler_params=pltpu.CompilerParams(collective_id=0),
    )(x, w_mat)
